# baseline (speedup 1.0000x reference)
import jax
import jax.numpy as jnp
from jax import lax
from jax.experimental import pallas as pl
from jax.experimental.pallas import tpu as pltpu
from jax.experimental.pallas import tpu_sc as plsc

N_NODES = 100000
D = 32
E = 1280000

NC, NS = 2, 16
W = NC * NS
EW = E // W
GE = 320
NG = EW // GE
SPG = 4
SR = GE // SPG

FOLD = 4
RB = 1000
DW = D // 2


def _normalize(emb128):
    def body(x_ref, o_ref):
        x = x_ref[...]
        r = lax.broadcasted_iota(jnp.int32, (FOLD * D, FOLD * D), 0) // D
        c = lax.broadcasted_iota(jnp.int32, (FOLD * D, FOLD * D), 1) // D
        m = (r == c).astype(jnp.float32)
        s = lax.dot_general(x * x, m, (((1,), (0,)), ((), ())),
                            preferred_element_type=jnp.float32)
        o_ref[...] = (x / jnp.maximum(jnp.sqrt(s), 1e-12)).astype(
            jnp.bfloat16).reshape(RB * FOLD * D)

    n128 = N_NODES // FOLD
    return pl.pallas_call(
        body,
        grid=(n128 // RB,),
        in_specs=[pl.BlockSpec((RB, FOLD * D), lambda i: (i, 0))],
        out_specs=pl.BlockSpec((RB * FOLD * D,), lambda i: (i,)),
        out_shape=jax.ShapeDtypeStruct((N_NODES * D,), jnp.bfloat16),
    )(emb128)


def _sc_body(table, sidx, didx, out, idx_v, src_v, dst_v, out_v, sem, isem):
    wid = lax.axis_index("s") * NC + lax.axis_index("c")
    ebase = wid * EW

    def idx_copy(g, slot):
        base = ebase + g * GE
        pltpu.async_copy(sidx.at[pl.ds(base, GE)], idx_v.at[slot, 0],
                         isem.at[slot])
        pltpu.async_copy(didx.at[pl.ds(base, GE)], idx_v.at[slot, 1],
                         isem.at[slot])

    def idx_wait(slot):
        pltpu.make_async_copy(sidx.at[pl.ds(0, GE)], idx_v.at[slot, 0],
                              isem.at[slot]).wait()
        pltpu.make_async_copy(didx.at[pl.ds(0, GE)], idx_v.at[slot, 1],
                              isem.at[slot]).wait()

    def fire(g, slot, buf):
        for j in range(SPG):
            pltpu.async_copy(table.at[idx_v.at[slot, 0, pl.ds(j * SR, SR)]],
                             src_v.at[buf, pl.ds(j * SR, SR)], sem.at[buf])
            pltpu.async_copy(table.at[idx_v.at[slot, 1, pl.ds(j * SR, SR)]],
                             dst_v.at[buf, pl.ds(j * SR, SR)], sem.at[buf])

    idx_copy(0, 0)
    idx_copy(1, 1)
    idx_wait(0)
    fire(0, 0, 0)
    lanes = lax.iota(jnp.int32, 16)
    cols = [(lanes + w) & (DW - 1) for w in range(DW)]

    def group_body(g, carry):
        cur = lax.rem(g, 2)
        nxt = 1 - cur

        @pl.when(g < NG - 1)
        def _():
            idx_wait(lax.rem(g + 1, 3))
            fire(g + 1, lax.rem(g + 1, 3), nxt)

        @pl.when(g < NG - 2)
        def _():
            idx_copy(g + 2, lax.rem(g + 2, 3))

        for j in range(2 * SPG):
            pltpu.make_async_copy(table.at[pl.ds(0, SR)],
                                  src_v.at[cur, pl.ds(0, SR)],
                                  sem.at[cur]).wait()

        sref = src_v.at[cur]
        dref = dst_v.at[cur]

        def batch_body(b, c):
            row = lanes + 16 * b
            acc = [jnp.zeros((16,), jnp.float32) for _ in range(4)]
            hmask = jnp.int32(-65536)
            for w in range(DW):
                sv = plsc.load_gather(sref, [row, cols[w]])
                dv = plsc.load_gather(dref, [row, cols[w]])
                slo = lax.bitcast_convert_type(sv << 16, jnp.float32)
                dlo = lax.bitcast_convert_type(dv << 16, jnp.float32)
                shi = lax.bitcast_convert_type(sv & hmask, jnp.float32)
                dhi = lax.bitcast_convert_type(dv & hmask, jnp.float32)
                acc[w % 4] = (acc[w % 4] + slo * dlo) + shi * dhi
            out_v[cur, pl.ds(16 * b, 16)] = (acc[0] + acc[1]) + (acc[2] + acc[3])
            return c

        lax.fori_loop(0, GE // 16, batch_body, 0)
        pltpu.sync_copy(out_v.at[cur], out.at[pl.ds(ebase + g * GE, GE)])
        return carry

    lax.fori_loop(0, NG, group_body, 0)


def kernel(embeddings, pos_edges, neg_edges):
    emb_n = _normalize(embeddings.reshape(N_NODES // FOLD, FOLD * D))
    sidx = jnp.concatenate([pos_edges[:, 0], neg_edges[:, 0]])
    didx = jnp.concatenate([pos_edges[:, 1], neg_edges[:, 1]])
    sc = pl.kernel(
        _sc_body,
        out_type=jax.ShapeDtypeStruct((E,), jnp.float32),
        mesh=plsc.VectorSubcoreMesh(core_axis_name="c", subcore_axis_name="s"),
        scratch_types=[
            pltpu.VMEM((3, 2, GE), jnp.int32),
            pltpu.VMEM((2, GE, DW), jnp.int32),
            pltpu.VMEM((2, GE, DW), jnp.int32),
            pltpu.VMEM((2, GE), jnp.float32),
            pltpu.SemaphoreType.DMA((2,)),
            pltpu.SemaphoreType.DMA((3,)),
        ],
        compiler_params=pltpu.CompilerParams(
            needs_layout_passes=False, use_tc_tiling_on_sc=False),
    )
    emb_bits = lax.bitcast_convert_type(
        emb_n.reshape(N_NODES, DW, 2), jnp.int32)
    return sc(emb_bits, sidx, didx)


# speedup vs baseline: 10.5446x; 10.5446x over previous
import jax
import jax.numpy as jnp
from jax import lax
from jax.experimental import pallas as pl
from jax.experimental.pallas import tpu as pltpu
from jax.experimental.pallas import tpu_sc as plsc

N_NODES = 100000
D = 32
E = 1280000

NC, NS = 2, 16
W = NC * NS
EW = E // W
GE = 320
NG = EW // GE
SPG = 4
SR = GE // SPG

FOLD = 4
RB = 1000


def _normalize(emb128):
    def body(x_ref, o_ref):
        x = x_ref[...]
        r = lax.broadcasted_iota(jnp.int32, (FOLD * D, FOLD * D), 0) // D
        c = lax.broadcasted_iota(jnp.int32, (FOLD * D, FOLD * D), 1) // D
        m = (r == c).astype(jnp.float32)
        s = lax.dot_general(x * x, m, (((1,), (0,)), ((), ())),
                            preferred_element_type=jnp.float32)
        o_ref[...] = (x / jnp.maximum(jnp.sqrt(s), 1e-12)).reshape(RB * FOLD * D)

    n128 = N_NODES // FOLD
    return pl.pallas_call(
        body,
        grid=(n128 // RB,),
        in_specs=[pl.BlockSpec((RB, FOLD * D), lambda i: (i, 0))],
        out_specs=pl.BlockSpec((RB * FOLD * D,), lambda i: (i,)),
        out_shape=jax.ShapeDtypeStruct((N_NODES * D,), jnp.float32),
    )(emb128)


def _sc_body(table, sidx, didx, out, idx_v, src_v, dst_v, out_v, sem, isem):
    wid = lax.axis_index("s") * NC + lax.axis_index("c")
    ebase = wid * EW

    def idx_copy(g, slot):
        base = ebase + g * GE
        pltpu.async_copy(sidx.at[pl.ds(base, GE)], idx_v.at[slot, 0],
                         isem.at[slot])
        pltpu.async_copy(didx.at[pl.ds(base, GE)], idx_v.at[slot, 1],
                         isem.at[slot])

    def idx_wait(slot):
        pltpu.make_async_copy(sidx.at[pl.ds(0, GE)], idx_v.at[slot, 0],
                              isem.at[slot]).wait()
        pltpu.make_async_copy(didx.at[pl.ds(0, GE)], idx_v.at[slot, 1],
                              isem.at[slot]).wait()

    def fire(g, slot, buf):
        for j in range(SPG):
            pltpu.async_copy(table.at[idx_v.at[slot, 0, pl.ds(j * SR, SR)]],
                             src_v.at[buf, pl.ds(j * SR, SR)], sem.at[buf])
            pltpu.async_copy(table.at[idx_v.at[slot, 1, pl.ds(j * SR, SR)]],
                             dst_v.at[buf, pl.ds(j * SR, SR)], sem.at[buf])

    idx_copy(0, 0)
    idx_copy(1, 1)
    idx_wait(0)
    fire(0, 0, 0)
    lanes = lax.iota(jnp.int32, 16)
    cols = [(lanes + d) & 31 for d in range(D)]

    def group_body(g, carry):
        cur = lax.rem(g, 2)
        nxt = 1 - cur

        @pl.when(g < NG - 1)
        def _():
            idx_wait(lax.rem(g + 1, 3))
            fire(g + 1, lax.rem(g + 1, 3), nxt)

        @pl.when(g < NG - 2)
        def _():
            idx_copy(g + 2, lax.rem(g + 2, 3))

        for j in range(2 * SPG):
            pltpu.make_async_copy(table.at[pl.ds(0, SR)],
                                  src_v.at[cur, pl.ds(0, SR)],
                                  sem.at[cur]).wait()

        sref = src_v.at[cur]
        dref = dst_v.at[cur]

        def batch_body(b, c):
            row = lanes + 16 * b
            acc = [jnp.zeros((16,), jnp.float32) for _ in range(4)]
            for d in range(D):
                col = cols[d]
                sv = plsc.load_gather(sref, [row, col])
                dv = plsc.load_gather(dref, [row, col])
                acc[d % 4] = acc[d % 4] + sv * dv
            out_v[cur, pl.ds(16 * b, 16)] = (acc[0] + acc[1]) + (acc[2] + acc[3])
            return c

        lax.fori_loop(0, GE // 16, batch_body, 0)
        pltpu.sync_copy(out_v.at[cur], out.at[pl.ds(ebase + g * GE, GE)])
        return carry

    lax.fori_loop(0, NG, group_body, 0)


def kernel(embeddings, pos_edges, neg_edges):
    emb_n = _normalize(embeddings.reshape(N_NODES // FOLD, FOLD * D))
    sidx = jnp.concatenate([pos_edges[:, 0], neg_edges[:, 0]])
    didx = jnp.concatenate([pos_edges[:, 1], neg_edges[:, 1]])
    sc = pl.kernel(
        _sc_body,
        out_type=jax.ShapeDtypeStruct((E,), jnp.float32),
        mesh=plsc.VectorSubcoreMesh(core_axis_name="c", subcore_axis_name="s"),
        scratch_types=[
            pltpu.VMEM((3, 2, GE), jnp.int32),
            pltpu.VMEM((2, GE, D), jnp.float32),
            pltpu.VMEM((2, GE, D), jnp.float32),
            pltpu.VMEM((2, GE), jnp.float32),
            pltpu.SemaphoreType.DMA((2,)),
            pltpu.SemaphoreType.DMA((3,)),
        ],
        compiler_params=pltpu.CompilerParams(
            needs_layout_passes=False, use_tc_tiling_on_sc=False),
    )
    return sc(emb_n.reshape(N_NODES, D), sidx, didx)


# speedup vs baseline: 10.8263x; 1.0267x over previous
import jax
import jax.numpy as jnp
from jax import lax
from jax.experimental import pallas as pl
from jax.experimental.pallas import tpu as pltpu
from jax.experimental.pallas import tpu_sc as plsc

N_NODES = 100000
D = 32
E = 1280000

NC, NS = 2, 16
W = NC * NS
EW = E // W
GE = 320
NG = EW // GE
SPG = 4
SR = GE // SPG

FOLD = 4
RB = 1000


def _normalize(emb128):
    def body(x_ref, o_ref):
        x = x_ref[...]
        r = lax.broadcasted_iota(jnp.int32, (FOLD * D, FOLD * D), 0) // D
        c = lax.broadcasted_iota(jnp.int32, (FOLD * D, FOLD * D), 1) // D
        m = (r == c).astype(jnp.float32)
        s = lax.dot_general(x * x, m, (((1,), (0,)), ((), ())),
                            preferred_element_type=jnp.float32)
        o_ref[...] = (x / jnp.maximum(jnp.sqrt(s), 1e-12)).reshape(RB * FOLD * D)

    n128 = N_NODES // FOLD
    return pl.pallas_call(
        body,
        grid=(n128 // RB,),
        in_specs=[pl.BlockSpec((RB, FOLD * D), lambda i: (i, 0))],
        out_specs=pl.BlockSpec((RB * FOLD * D,), lambda i: (i,)),
        out_shape=jax.ShapeDtypeStruct((N_NODES * D,), jnp.float32),
    )(emb128)


def _sc_body(table, sidx, didx, out, idx_v, src_v, dst_v, out_v, sem, isem):
    wid = lax.axis_index("s") * NC + lax.axis_index("c")
    ebase = wid * EW

    def idx_copy(g, slot):
        base = ebase + g * GE
        pltpu.async_copy(sidx.at[pl.ds(base, GE)], idx_v.at[slot, 0],
                         isem.at[slot])
        pltpu.async_copy(didx.at[pl.ds(base, GE)], idx_v.at[slot, 1],
                         isem.at[slot])

    def idx_wait(slot):
        pltpu.make_async_copy(sidx.at[pl.ds(0, GE)], idx_v.at[slot, 0],
                              isem.at[slot]).wait()
        pltpu.make_async_copy(didx.at[pl.ds(0, GE)], idx_v.at[slot, 1],
                              isem.at[slot]).wait()

    def fire(g, slot, buf):
        for j in range(SPG):
            pltpu.async_copy(table.at[idx_v.at[slot, 0, pl.ds(j * SR, SR)]],
                             src_v.at[buf, pl.ds(j * SR, SR)], sem.at[buf, j])
            pltpu.async_copy(table.at[idx_v.at[slot, 1, pl.ds(j * SR, SR)]],
                             dst_v.at[buf, pl.ds(j * SR, SR)], sem.at[buf, j])

    idx_copy(0, 0)
    idx_copy(1, 1)
    idx_wait(0)
    fire(0, 0, 0)
    lanes = lax.iota(jnp.int32, 16)
    cols = [(lanes + d) & 31 for d in range(D)]

    def group_body(g, carry):
        cur = lax.rem(g, 2)
        nxt = 1 - cur

        @pl.when(g < NG - 1)
        def _():
            idx_wait(lax.rem(g + 1, 3))
            fire(g + 1, lax.rem(g + 1, 3), nxt)

        @pl.when(g < NG - 2)
        def _():
            idx_copy(g + 2, lax.rem(g + 2, 3))

        sref = src_v.at[cur]
        dref = dst_v.at[cur]

        def batch_body(b, c):
            row = lanes + 16 * b
            acc = [jnp.zeros((16,), jnp.float32) for _ in range(4)]
            for d in range(D):
                col = cols[d]
                sv = plsc.load_gather(sref, [row, col])
                dv = plsc.load_gather(dref, [row, col])
                acc[d % 4] = acc[d % 4] + sv * dv
            out_v[cur, pl.ds(16 * b, 16)] = (acc[0] + acc[1]) + (acc[2] + acc[3])
            return c

        for j in range(SPG):
            for _ in range(2):
                pltpu.make_async_copy(table.at[pl.ds(0, SR)],
                                      src_v.at[cur, pl.ds(j * SR, SR)],
                                      sem.at[cur, j]).wait()
            lax.fori_loop(j * SR // 16, (j + 1) * SR // 16, batch_body, 0)
        pltpu.sync_copy(out_v.at[cur], out.at[pl.ds(ebase + g * GE, GE)])
        return carry

    lax.fori_loop(0, NG, group_body, 0)


def kernel(embeddings, pos_edges, neg_edges):
    emb_n = _normalize(embeddings.reshape(N_NODES // FOLD, FOLD * D))
    sidx = jnp.concatenate([pos_edges[:, 0], neg_edges[:, 0]])
    didx = jnp.concatenate([pos_edges[:, 1], neg_edges[:, 1]])
    sc = pl.kernel(
        _sc_body,
        out_type=jax.ShapeDtypeStruct((E,), jnp.float32),
        mesh=plsc.VectorSubcoreMesh(core_axis_name="c", subcore_axis_name="s"),
        scratch_types=[
            pltpu.VMEM((3, 2, GE), jnp.int32),
            pltpu.VMEM((2, GE, D), jnp.float32),
            pltpu.VMEM((2, GE, D), jnp.float32),
            pltpu.VMEM((2, GE), jnp.float32),
            pltpu.SemaphoreType.DMA((2, SPG)),
            pltpu.SemaphoreType.DMA((3,)),
        ],
        compiler_params=pltpu.CompilerParams(
            needs_layout_passes=False, use_tc_tiling_on_sc=False),
    )
    return sc(emb_n.reshape(N_NODES, D), sidx, didx)
